# Initial kernel scaffold; baseline (speedup 1.0000x reference)
import jax
import jax.numpy as jnp
from jax.experimental import pallas as pl


def kernel(features, edge_index, W, b):
    raise NotImplementedError("write your pallas kernel here")


# speedup vs baseline: 8.7676x; 8.7676x over previous
import functools

import jax
import jax.numpy as jnp
from jax import lax
from jax.experimental import pallas as pl
from jax.experimental.pallas import tpu as pltpu
from jax.experimental.pallas import tpu_sc as plsc

NC = 2
NS = 16
CHUNK = 80


def _sc_segment_sum(f_lo, f_hi, edge_r, z_feat, z_cnt, ones8, V, H, n_chunks):
    r_stride, r_len = 624, 640
    assert (NS - 1) * r_stride + r_len == V
    mesh = plsc.VectorSubcoreMesh(core_axis_name="c", subcore_axis_name="s")

    @functools.partial(
        pl.kernel,
        mesh=mesh,
        compiler_params=pltpu.CompilerParams(use_tc_tiling_on_sc=False),
        out_type=(
            jax.ShapeDtypeStruct((V, H), jnp.float32),
            jax.ShapeDtypeStruct((V, H), jnp.float32),
            jax.ShapeDtypeStruct((V, 8), jnp.float32),
        ),
        scratch_types=[
            pltpu.VMEM((n_chunks, CHUNK), jnp.int32),
            pltpu.VMEM((n_chunks, CHUNK), jnp.int32),
            pltpu.VMEM((CHUNK, H), jnp.float32),
            pltpu.VMEM((CHUNK, 8), jnp.float32),
            pltpu.VMEM_SHARED((V, H), jnp.float32),
            pltpu.VMEM_SHARED((V, 8), jnp.float32),
            pltpu.SemaphoreType.DMA,
        ],
    )
    def sc_kernel(flo_hbm, fhi_hbm, edge_hbm, zf_hbm, zc_hbm, ones_hbm,
                  sum_lo_out, sum_hi_out, counts_out,
                  src_v, tgt_v, rows_v, ones_v, acc_sh, cnt_sh, sem):
        c = lax.axis_index("c")
        s = lax.axis_index("s")
        r0 = s * r_stride

        pltpu.sync_copy(zf_hbm.at[pl.ds(r0, r_len)],
                        acc_sh.at[pl.ds(r0, r_len)])
        pltpu.sync_copy(edge_hbm.at[0, s], src_v)
        pltpu.sync_copy(edge_hbm.at[1, s], tgt_v)

        @pl.when(c == 0)
        def _():
            pltpu.sync_copy(zc_hbm.at[pl.ds(r0, r_len)],
                            cnt_sh.at[pl.ds(r0, r_len)])
            pltpu.sync_copy(ones_hbm, ones_v)

        plsc.subcore_barrier()

        def body(i, carry):
            @pl.when(c == 0)
            def _():
                pltpu.async_copy(flo_hbm.at[src_v.at[i]], rows_v, sem).wait()
                pltpu.sync_copy(ones_v, cnt_sh.at[tgt_v.at[i]], add=True)

            @pl.when(c == 1)
            def _():
                pltpu.async_copy(fhi_hbm.at[src_v.at[i]], rows_v, sem).wait()

            pltpu.sync_copy(rows_v, acc_sh.at[tgt_v.at[i]], add=True)
            return carry

        lax.fori_loop(0, n_chunks, body, 0)

        plsc.subcore_barrier()

        @pl.when(c == 0)
        def _():
            pltpu.sync_copy(acc_sh.at[pl.ds(r0, r_len)],
                            sum_lo_out.at[pl.ds(r0, r_len)])
            pltpu.sync_copy(cnt_sh.at[pl.ds(r0, r_len)],
                            counts_out.at[pl.ds(r0, r_len)])

        @pl.when(c == 1)
        def _():
            pltpu.sync_copy(acc_sh.at[pl.ds(r0, r_len)],
                            sum_hi_out.at[pl.ds(r0, r_len)])

    return sc_kernel(f_lo, f_hi, edge_r, z_feat, z_cnt, ones8)


def _tc_combine(features, sum_lo, sum_hi, counts, A, B2L, B2H, bias, V, C, H):
    BLK = 1000
    grid = (V // BLK,)

    def body(f_ref, sl_ref, sh_ref, c_ref, a_ref, b2l_ref, b2h_ref,
             bias_ref, o_ref):
        cnt = c_ref[:, 0:1]
        mask = (cnt > 0.0).astype(jnp.float32)
        inv = 1.0 / jnp.maximum(cnt, 1.0)
        local = jnp.dot(f_ref[...], a_ref[...],
                        preferred_element_type=jnp.float32)
        nbr = (jnp.dot(sl_ref[...] * inv, b2l_ref[...],
                       preferred_element_type=jnp.float32)
               + jnp.dot(sh_ref[...] * inv, b2h_ref[...],
                         preferred_element_type=jnp.float32))
        o_ref[...] = mask * (local + bias_ref[...]) + nbr

    return pl.pallas_call(
        body,
        grid=grid,
        in_specs=[
            pl.BlockSpec((BLK, C), lambda i: (i, 0)),
            pl.BlockSpec((BLK, H), lambda i: (i, 0)),
            pl.BlockSpec((BLK, H), lambda i: (i, 0)),
            pl.BlockSpec((BLK, 8), lambda i: (i, 0)),
            pl.BlockSpec((C, C), lambda i: (0, 0)),
            pl.BlockSpec((H, C), lambda i: (0, 0)),
            pl.BlockSpec((H, C), lambda i: (0, 0)),
            pl.BlockSpec((1, C), lambda i: (0, 0)),
        ],
        out_specs=pl.BlockSpec((BLK, C), lambda i: (i, 0)),
        out_shape=jax.ShapeDtypeStruct((V, C), jnp.float32),
    )(features, sum_lo, sum_hi, counts, A, B2L, B2H, bias)


def kernel(features, edge_index, W, b):
    V, C = features.shape
    H = C // 2
    E = edge_index.shape[1]
    n_chunks = E // (NS * CHUNK)
    edge_r = edge_index.reshape(2, NS, n_chunks, CHUNK)
    f_lo = features[:, :H]
    f_hi = features[:, H:]
    z_feat = jnp.zeros((V, H), jnp.float32)
    z_cnt = jnp.zeros((V, 8), jnp.float32)
    ones8 = jnp.ones((CHUNK, 8), jnp.float32)
    sum_lo, sum_hi, counts = _sc_segment_sum(
        f_lo, f_hi, edge_r, z_feat, z_cnt, ones8, V, H, n_chunks)
    W1 = W[:, :C]
    W2 = W[:, C:]
    A = (W1 - W2).T
    B2 = W2.T
    return _tc_combine(features, sum_lo, sum_hi, counts,
                       A, B2[:H], B2[H:], b.reshape(1, C), V, C, H)


# speedup vs baseline: 11.2132x; 1.2789x over previous
import functools

import jax
import jax.numpy as jnp
from jax import lax
from jax.experimental import pallas as pl
from jax.experimental.pallas import tpu as pltpu
from jax.experimental.pallas import tpu_sc as plsc

NC = 2
NS = 16
CHUNK = 80


def _sc_segment_sum(f_lo, f_hi, edge_r, z_feat, z_cnt, ones8, V, H, n_chunks):
    r_stride, r_len = 624, 640
    assert (NS - 1) * r_stride + r_len == V
    mesh = plsc.VectorSubcoreMesh(core_axis_name="c", subcore_axis_name="s")

    @functools.partial(
        pl.kernel,
        mesh=mesh,
        compiler_params=pltpu.CompilerParams(use_tc_tiling_on_sc=False),
        out_type=(
            jax.ShapeDtypeStruct((V, H), jnp.float32),
            jax.ShapeDtypeStruct((V, H), jnp.float32),
            jax.ShapeDtypeStruct((V, 8), jnp.float32),
        ),
        scratch_types=[
            pltpu.VMEM((n_chunks, CHUNK), jnp.int32),
            pltpu.VMEM((n_chunks, CHUNK), jnp.int32),
            pltpu.VMEM((CHUNK, H), jnp.float32),
            pltpu.VMEM((CHUNK, H), jnp.float32),
            pltpu.VMEM((CHUNK, 8), jnp.float32),
            pltpu.VMEM_SHARED((V, H), jnp.float32),
            pltpu.VMEM_SHARED((V, 8), jnp.float32),
            pltpu.SemaphoreType.DMA,
            pltpu.SemaphoreType.DMA,
            pltpu.SemaphoreType.DMA,
            pltpu.SemaphoreType.DMA,
            pltpu.SemaphoreType.DMA,
            pltpu.SemaphoreType.DMA,
        ],
    )
    def sc_kernel(flo_hbm, fhi_hbm, edge_hbm, zf_hbm, zc_hbm, ones_hbm,
                  sum_lo_out, sum_hi_out, counts_out,
                  src_v, tgt_v, rows0, rows1, ones_v, acc_sh, cnt_sh,
                  gsem0, gsem1, ssem0, ssem1, csem0, csem1):
        c = lax.axis_index("c")
        s = lax.axis_index("s")
        r0 = s * r_stride

        pltpu.sync_copy(zf_hbm.at[pl.ds(r0, r_len)],
                        acc_sh.at[pl.ds(r0, r_len)])
        pltpu.sync_copy(edge_hbm.at[0, s], src_v)
        pltpu.sync_copy(edge_hbm.at[1, s], tgt_v)

        @pl.when(c == 0)
        def _():
            pltpu.sync_copy(zc_hbm.at[pl.ds(r0, r_len)],
                            cnt_sh.at[pl.ds(r0, r_len)])
            pltpu.sync_copy(ones_hbm, ones_v)

        plsc.subcore_barrier()

        def start_gather(i, buf, sem):
            @pl.when(c == 0)
            def _():
                pltpu.async_copy(flo_hbm.at[src_v.at[i]], buf, sem)

            @pl.when(c == 1)
            def _():
                pltpu.async_copy(fhi_hbm.at[src_v.at[i]], buf, sem)

        def wait_gather(buf, sem):
            pltpu.make_async_copy(zf_hbm.at[pl.ds(0, CHUNK)], buf, sem).wait()

        def start_scatter(i, buf, sem):
            pltpu.async_copy(buf, acc_sh.at[tgt_v.at[i]], sem, add=True)

        def wait_scatter(buf, sem):
            pltpu.make_async_copy(buf, acc_sh.at[pl.ds(0, CHUNK)], sem).wait()

        def start_counts(i, sem):
            @pl.when(c == 0)
            def _():
                pltpu.async_copy(ones_v, cnt_sh.at[tgt_v.at[i]], sem,
                                 add=True)

        def wait_counts(sem):
            @pl.when(c == 0)
            def _():
                pltpu.make_async_copy(ones_v, cnt_sh.at[pl.ds(0, CHUNK)],
                                      sem).wait()

        n_pairs = n_chunks // 2
        start_gather(0, rows0, gsem0)

        def pair_body(g, carry):
            i0 = 2 * g
            i1 = i0 + 1
            wait_gather(rows0, gsem0)

            @pl.when(g > 0)
            def _():
                wait_scatter(rows1, ssem1)
                wait_counts(csem0)

            start_scatter(i0, rows0, ssem0)
            start_counts(i0, csem0)
            start_gather(i1, rows1, gsem1)

            wait_gather(rows1, gsem1)
            wait_scatter(rows0, ssem0)

            @pl.when(g > 0)
            def _():
                wait_counts(csem1)

            start_scatter(i1, rows1, ssem1)
            start_counts(i1, csem1)

            @pl.when(g < n_pairs - 1)
            def _():
                start_gather(i0 + 2, rows0, gsem0)

            return carry

        lax.fori_loop(0, n_pairs, pair_body, 0)

        wait_scatter(rows1, ssem1)
        wait_counts(csem0)
        wait_counts(csem1)

        plsc.subcore_barrier()

        @pl.when(c == 0)
        def _():
            pltpu.sync_copy(acc_sh.at[pl.ds(r0, r_len)],
                            sum_lo_out.at[pl.ds(r0, r_len)])
            pltpu.sync_copy(cnt_sh.at[pl.ds(r0, r_len)],
                            counts_out.at[pl.ds(r0, r_len)])

        @pl.when(c == 1)
        def _():
            pltpu.sync_copy(acc_sh.at[pl.ds(r0, r_len)],
                            sum_hi_out.at[pl.ds(r0, r_len)])

    return sc_kernel(f_lo, f_hi, edge_r, z_feat, z_cnt, ones8)


def _tc_combine(features, sum_lo, sum_hi, counts, A, B2L, B2H, bias, V, C, H):
    BLK = 1000
    grid = (V // BLK,)

    def body(f_ref, sl_ref, sh_ref, c_ref, a_ref, b2l_ref, b2h_ref,
             bias_ref, o_ref):
        cnt = c_ref[:, 0:1]
        mask = (cnt > 0.0).astype(jnp.float32)
        inv = 1.0 / jnp.maximum(cnt, 1.0)
        local = jnp.dot(f_ref[...], a_ref[...],
                        preferred_element_type=jnp.float32)
        nbr = (jnp.dot(sl_ref[...] * inv, b2l_ref[...],
                       preferred_element_type=jnp.float32)
               + jnp.dot(sh_ref[...] * inv, b2h_ref[...],
                         preferred_element_type=jnp.float32))
        o_ref[...] = mask * (local + bias_ref[...]) + nbr

    return pl.pallas_call(
        body,
        grid=grid,
        in_specs=[
            pl.BlockSpec((BLK, C), lambda i: (i, 0)),
            pl.BlockSpec((BLK, H), lambda i: (i, 0)),
            pl.BlockSpec((BLK, H), lambda i: (i, 0)),
            pl.BlockSpec((BLK, 8), lambda i: (i, 0)),
            pl.BlockSpec((C, C), lambda i: (0, 0)),
            pl.BlockSpec((H, C), lambda i: (0, 0)),
            pl.BlockSpec((H, C), lambda i: (0, 0)),
            pl.BlockSpec((1, C), lambda i: (0, 0)),
        ],
        out_specs=pl.BlockSpec((BLK, C), lambda i: (i, 0)),
        out_shape=jax.ShapeDtypeStruct((V, C), jnp.float32),
    )(features, sum_lo, sum_hi, counts, A, B2L, B2H, bias)


def kernel(features, edge_index, W, b):
    V, C = features.shape
    H = C // 2
    E = edge_index.shape[1]
    n_chunks = E // (NS * CHUNK)
    edge_r = edge_index.reshape(2, NS, n_chunks, CHUNK)
    f_lo = features[:, :H]
    f_hi = features[:, H:]
    z_feat = jnp.zeros((V, H), jnp.float32)
    z_cnt = jnp.zeros((V, 8), jnp.float32)
    ones8 = jnp.ones((CHUNK, 8), jnp.float32)
    sum_lo, sum_hi, counts = _sc_segment_sum(
        f_lo, f_hi, edge_r, z_feat, z_cnt, ones8, V, H, n_chunks)
    W1 = W[:, :C]
    W2 = W[:, C:]
    A = (W1 - W2).T
    B2 = W2.T
    return _tc_combine(features, sum_lo, sum_hi, counts,
                       A, B2[:H], B2[H:], b.reshape(1, C), V, C, H)


# speedup vs baseline: 19.5008x; 1.7391x over previous
import functools

import jax
import jax.numpy as jnp
from jax import lax
from jax.experimental import pallas as pl
from jax.experimental.pallas import tpu as pltpu
from jax.experimental.pallas import tpu_sc as plsc

NC = 2
NS = 16
CHUNK = 80


def _sc_segment_sum(f_lo, f_hi, edge_r, z_feat, z_cnt, ones8, V, H, n_chunks):
    r_stride, r_len = 624, 640
    assert (NS - 1) * r_stride + r_len == V
    mesh = plsc.VectorSubcoreMesh(core_axis_name="c", subcore_axis_name="s")

    @functools.partial(
        pl.kernel,
        mesh=mesh,
        compiler_params=pltpu.CompilerParams(use_tc_tiling_on_sc=False),
        out_type=(
            jax.ShapeDtypeStruct((V, H), jnp.float32),
            jax.ShapeDtypeStruct((V, H), jnp.float32),
            jax.ShapeDtypeStruct((V, 8), jnp.float32),
        ),
        scratch_types=[
            pltpu.VMEM((n_chunks, CHUNK), jnp.int32),
            pltpu.VMEM((n_chunks, CHUNK), jnp.int32),
            pltpu.VMEM((CHUNK, H), jnp.float32),
            pltpu.VMEM((CHUNK, H), jnp.float32),
            pltpu.VMEM((CHUNK, H), jnp.float32),
            pltpu.VMEM((CHUNK, H), jnp.float32),
            pltpu.VMEM((CHUNK, 8), jnp.float32),
            pltpu.VMEM_SHARED((V, H), jnp.float32),
            pltpu.VMEM_SHARED((V, 8), jnp.float32),
        ] + [pltpu.SemaphoreType.DMA] * 12,
    )
    def sc_kernel(flo_hbm, fhi_hbm, edge_hbm, zf_hbm, zc_hbm, ones_hbm,
                  sum_lo_out, sum_hi_out, counts_out,
                  src_v, tgt_v, rows0, rows1, rows2, rows3, ones_v,
                  acc_sh, cnt_sh, *sems):
        c = lax.axis_index("c")
        s = lax.axis_index("s")
        r0 = s * r_stride

        pltpu.sync_copy(zf_hbm.at[pl.ds(r0, r_len)],
                        acc_sh.at[pl.ds(r0, r_len)])
        pltpu.sync_copy(edge_hbm.at[0, s], src_v)
        pltpu.sync_copy(edge_hbm.at[1, s], tgt_v)

        @pl.when(c == 0)
        def _():
            pltpu.sync_copy(zc_hbm.at[pl.ds(r0, r_len)],
                            cnt_sh.at[pl.ds(r0, r_len)])
            pltpu.sync_copy(ones_hbm, ones_v)

        plsc.subcore_barrier()

        bufs = (rows0, rows1, rows2, rows3)
        gsems, ssems, csems = sems[0:4], sems[4:8], sems[8:12]

        def start_gather(i, buf, sem):
            @pl.when(c == 0)
            def _():
                pltpu.async_copy(flo_hbm.at[src_v.at[i]], buf, sem)

            @pl.when(c == 1)
            def _():
                pltpu.async_copy(fhi_hbm.at[src_v.at[i]], buf, sem)

        def wait_gather(buf, sem):
            pltpu.make_async_copy(zf_hbm.at[pl.ds(0, CHUNK)], buf, sem).wait()

        def start_scatter(i, buf, sem):
            pltpu.async_copy(buf, acc_sh.at[tgt_v.at[i]], sem, add=True)

        def wait_scatter(buf, sem):
            pltpu.make_async_copy(buf, acc_sh.at[pl.ds(0, CHUNK)], sem).wait()

        def start_counts(i, sem):
            @pl.when(c == 0)
            def _():
                pltpu.async_copy(ones_v, cnt_sh.at[tgt_v.at[i]], sem,
                                 add=True)

        def wait_counts(sem):
            @pl.when(c == 0)
            def _():
                pltpu.make_async_copy(ones_v, cnt_sh.at[pl.ds(0, CHUNK)],
                                      sem).wait()

        def chunk_step(j, p):
            wait_gather(bufs[p], gsems[p])
            start_scatter(j, bufs[p], ssems[p])
            start_counts(j, csems[p])
            p3 = (p + 3) % 4

            @pl.when(j + 3 < n_chunks)
            def _():
                wait_scatter(bufs[p3], ssems[p3])
                wait_counts(csems[p3])
                start_gather(j + 3, bufs[p3], gsems[p3])

        for j in range(3):
            start_gather(j, bufs[j], gsems[j])
        wait_gather(bufs[0], gsems[0])
        start_scatter(0, bufs[0], ssems[0])
        start_counts(0, csems[0])
        start_gather(3, bufs[3], gsems[3])

        def quad_body(q, carry):
            j = 4 * q + 1
            for t in range(4):
                chunk_step(j + t, (1 + t) % 4)
            return carry

        lax.fori_loop(0, (n_chunks - 2) // 4, quad_body, 0)
        chunk_step(jnp.int32(n_chunks - 1), (n_chunks - 1) % 4)

        for p in range(4):
            wait_scatter(bufs[p], ssems[p])
            wait_counts(csems[p])

        plsc.subcore_barrier()

        @pl.when(c == 0)
        def _():
            pltpu.sync_copy(acc_sh.at[pl.ds(r0, r_len)],
                            sum_lo_out.at[pl.ds(r0, r_len)])
            pltpu.sync_copy(cnt_sh.at[pl.ds(r0, r_len)],
                            counts_out.at[pl.ds(r0, r_len)])

        @pl.when(c == 1)
        def _():
            pltpu.sync_copy(acc_sh.at[pl.ds(r0, r_len)],
                            sum_hi_out.at[pl.ds(r0, r_len)])

    return sc_kernel(f_lo, f_hi, edge_r, z_feat, z_cnt, ones8)


def _tc_combine(features, sum_lo, sum_hi, counts, A, B2L, B2H, bias, V, C, H):
    BLK = 1000
    grid = (V // BLK,)

    def body(f_ref, sl_ref, sh_ref, c_ref, a_ref, b2l_ref, b2h_ref,
             bias_ref, o_ref):
        cnt = c_ref[:, 0:1]
        mask = (cnt > 0.0).astype(jnp.float32)
        inv = 1.0 / jnp.maximum(cnt, 1.0)
        local = jnp.dot(f_ref[...], a_ref[...],
                        preferred_element_type=jnp.float32)
        nbr = (jnp.dot(sl_ref[...] * inv, b2l_ref[...],
                       preferred_element_type=jnp.float32)
               + jnp.dot(sh_ref[...] * inv, b2h_ref[...],
                         preferred_element_type=jnp.float32))
        o_ref[...] = mask * (local + bias_ref[...]) + nbr

    return pl.pallas_call(
        body,
        grid=grid,
        in_specs=[
            pl.BlockSpec((BLK, C), lambda i: (i, 0)),
            pl.BlockSpec((BLK, H), lambda i: (i, 0)),
            pl.BlockSpec((BLK, H), lambda i: (i, 0)),
            pl.BlockSpec((BLK, 8), lambda i: (i, 0)),
            pl.BlockSpec((C, C), lambda i: (0, 0)),
            pl.BlockSpec((H, C), lambda i: (0, 0)),
            pl.BlockSpec((H, C), lambda i: (0, 0)),
            pl.BlockSpec((1, C), lambda i: (0, 0)),
        ],
        out_specs=pl.BlockSpec((BLK, C), lambda i: (i, 0)),
        out_shape=jax.ShapeDtypeStruct((V, C), jnp.float32),
    )(features, sum_lo, sum_hi, counts, A, B2L, B2H, bias)


def kernel(features, edge_index, W, b):
    V, C = features.shape
    H = C // 2
    E = edge_index.shape[1]
    n_chunks = E // (NS * CHUNK)
    edge_r = edge_index.reshape(2, NS, n_chunks, CHUNK)
    f_lo = features[:, :H]
    f_hi = features[:, H:]
    z_feat = jnp.zeros((V, H), jnp.float32)
    z_cnt = jnp.zeros((V, 8), jnp.float32)
    ones8 = jnp.ones((CHUNK, 8), jnp.float32)
    sum_lo, sum_hi, counts = _sc_segment_sum(
        f_lo, f_hi, edge_r, z_feat, z_cnt, ones8, V, H, n_chunks)
    W1 = W[:, :C]
    W2 = W[:, C:]
    A = (W1 - W2).T
    B2 = W2.T
    return _tc_combine(features, sum_lo, sum_hi, counts,
                       A, B2[:H], B2[H:], b.reshape(1, C), V, C, H)


# speedup vs baseline: 20.9649x; 1.0751x over previous
import functools

import jax
import jax.numpy as jnp
from jax import lax
from jax.experimental import pallas as pl
from jax.experimental.pallas import tpu as pltpu
from jax.experimental.pallas import tpu_sc as plsc

NC = 2
NS = 16
CHUNK = 80


def _sc_segment_sum(feats2, edge_r, z_feat, z_cnt, ones8, V, H, n_chunks):
    r_stride, r_len = 624, 640
    assert (NS - 1) * r_stride + r_len == V
    mesh = plsc.VectorSubcoreMesh(core_axis_name="c", subcore_axis_name="s")

    @functools.partial(
        pl.kernel,
        mesh=mesh,
        compiler_params=pltpu.CompilerParams(use_tc_tiling_on_sc=False),
        out_type=(
            jax.ShapeDtypeStruct((V, H), jnp.float32),
            jax.ShapeDtypeStruct((V, H), jnp.float32),
            jax.ShapeDtypeStruct((V, 8), jnp.float32),
        ),
        scratch_types=[
            pltpu.VMEM((n_chunks, CHUNK), jnp.int32),
            pltpu.VMEM((n_chunks, CHUNK), jnp.int32),
            pltpu.VMEM((CHUNK,), jnp.int32),
            pltpu.VMEM((CHUNK,), jnp.int32),
            pltpu.VMEM((CHUNK,), jnp.int32),
            pltpu.VMEM((CHUNK,), jnp.int32),
            pltpu.VMEM((CHUNK, H), jnp.float32),
            pltpu.VMEM((CHUNK, H), jnp.float32),
            pltpu.VMEM((CHUNK, H), jnp.float32),
            pltpu.VMEM((CHUNK, H), jnp.float32),
            pltpu.VMEM((CHUNK, 8), jnp.float32),
            pltpu.VMEM_SHARED((V, H), jnp.float32),
            pltpu.VMEM_SHARED((V, 8), jnp.float32),
        ] + [pltpu.SemaphoreType.DMA] * 12,
    )
    def sc_kernel(f2_hbm, edge_hbm, zf_hbm, zc_hbm, ones_hbm,
                  sum_lo_out, sum_hi_out, counts_out,
                  src_v, tgt_v, idx0, idx1, idx2, idx3,
                  rows0, rows1, rows2, rows3, ones_v,
                  acc_sh, cnt_sh, *sems):
        c = lax.axis_index("c")
        s = lax.axis_index("s")
        r0 = s * r_stride

        pltpu.sync_copy(zf_hbm, acc_sh.at[pl.ds(r0, r_len)])
        pltpu.sync_copy(edge_hbm.at[0, s], src_v)
        pltpu.sync_copy(edge_hbm.at[1, s], tgt_v)

        @pl.when(c == 0)
        def _():
            pltpu.sync_copy(zc_hbm, cnt_sh.at[pl.ds(r0, r_len)])
            pltpu.sync_copy(ones_hbm, ones_v)

        plsc.subcore_barrier()

        bufs = (rows0, rows1, rows2, rows3)
        ibufs = (idx0, idx1, idx2, idx3)
        gsems, ssems, csems = sems[0:4], sems[4:8], sems[8:12]

        def start_gather(i, buf, sem, ib):
            for k in range(CHUNK // 16):
                sl = pl.ds(16 * k, 16)
                ib[sl] = src_v[i, sl] * 2 + c
            pltpu.async_copy(f2_hbm.at[ib], buf, sem)

        def wait_gather(buf, sem):
            pltpu.make_async_copy(f2_hbm.at[pl.ds(0, CHUNK)], buf, sem).wait()

        def start_scatter(i, buf, sem):
            pltpu.async_copy(buf, acc_sh.at[tgt_v.at[i]], sem, add=True)

        def wait_scatter(buf, sem):
            pltpu.make_async_copy(buf, acc_sh.at[pl.ds(0, CHUNK)], sem).wait()

        def start_counts(i, sem):
            @pl.when(c == 0)
            def _():
                pltpu.async_copy(ones_v, cnt_sh.at[tgt_v.at[i]], sem,
                                 add=True)

        def wait_counts(sem):
            @pl.when(c == 0)
            def _():
                pltpu.make_async_copy(ones_v, cnt_sh.at[pl.ds(0, CHUNK)],
                                      sem).wait()

        def chunk_step(j, p):
            wait_gather(bufs[p], gsems[p])
            start_scatter(j, bufs[p], ssems[p])
            start_counts(j, csems[p])
            p3 = (p + 3) % 4

            @pl.when(j + 3 < n_chunks)
            def _():
                wait_scatter(bufs[p3], ssems[p3])
                wait_counts(csems[p3])
                start_gather(j + 3, bufs[p3], gsems[p3], ibufs[p3])

        for j in range(3):
            start_gather(j, bufs[j], gsems[j], ibufs[j])
        wait_gather(bufs[0], gsems[0])
        start_scatter(0, bufs[0], ssems[0])
        start_counts(0, csems[0])
        start_gather(3, bufs[3], gsems[3], ibufs[3])

        def quad_body(q, carry):
            j = 4 * q + 1
            for t in range(4):
                chunk_step(j + t, (1 + t) % 4)
            return carry

        lax.fori_loop(0, (n_chunks - 2) // 4, quad_body, 0)
        chunk_step(jnp.int32(n_chunks - 1), (n_chunks - 1) % 4)

        for p in range(4):
            wait_scatter(bufs[p], ssems[p])
            wait_counts(csems[p])

        plsc.subcore_barrier()

        @pl.when(c == 0)
        def _():
            pltpu.sync_copy(acc_sh.at[pl.ds(r0, r_len)],
                            sum_lo_out.at[pl.ds(r0, r_len)])
            pltpu.sync_copy(cnt_sh.at[pl.ds(r0, r_len)],
                            counts_out.at[pl.ds(r0, r_len)])

        @pl.when(c == 1)
        def _():
            pltpu.sync_copy(acc_sh.at[pl.ds(r0, r_len)],
                            sum_hi_out.at[pl.ds(r0, r_len)])

    return sc_kernel(feats2, edge_r, z_feat, z_cnt, ones8)


def _tc_combine(features, sum_lo, sum_hi, counts, A, B2L, B2H, bias, V, C, H):
    BLK = 1000
    grid = (V // BLK,)

    def body(f_ref, sl_ref, sh_ref, c_ref, a_ref, b2l_ref, b2h_ref,
             bias_ref, o_ref):
        cnt = c_ref[:, 0:1]
        mask = (cnt > 0.0).astype(jnp.float32)
        inv = 1.0 / jnp.maximum(cnt, 1.0)
        local = jnp.dot(f_ref[...], a_ref[...],
                        preferred_element_type=jnp.float32)
        nbr = (jnp.dot(sl_ref[...] * inv, b2l_ref[...],
                       preferred_element_type=jnp.float32)
               + jnp.dot(sh_ref[...] * inv, b2h_ref[...],
                         preferred_element_type=jnp.float32))
        o_ref[...] = mask * (local + bias_ref[...]) + nbr

    return pl.pallas_call(
        body,
        grid=grid,
        in_specs=[
            pl.BlockSpec((BLK, C), lambda i: (i, 0)),
            pl.BlockSpec((BLK, H), lambda i: (i, 0)),
            pl.BlockSpec((BLK, H), lambda i: (i, 0)),
            pl.BlockSpec((BLK, 8), lambda i: (i, 0)),
            pl.BlockSpec((C, C), lambda i: (0, 0)),
            pl.BlockSpec((H, C), lambda i: (0, 0)),
            pl.BlockSpec((H, C), lambda i: (0, 0)),
            pl.BlockSpec((1, C), lambda i: (0, 0)),
        ],
        out_specs=pl.BlockSpec((BLK, C), lambda i: (i, 0)),
        out_shape=jax.ShapeDtypeStruct((V, C), jnp.float32),
    )(features, sum_lo, sum_hi, counts, A, B2L, B2H, bias)


def kernel(features, edge_index, W, b):
    V, C = features.shape
    H = C // 2
    E = edge_index.shape[1]
    n_chunks = E // (NS * CHUNK)
    edge_r = edge_index.reshape(2, NS, n_chunks, CHUNK)
    feats2 = features.reshape(2 * V, H)
    z_feat = jnp.zeros((640, H), jnp.float32)
    z_cnt = jnp.zeros((640, 8), jnp.float32)
    ones8 = jnp.ones((CHUNK, 8), jnp.float32)
    sum_lo, sum_hi, counts = _sc_segment_sum(
        feats2, edge_r, z_feat, z_cnt, ones8, V, H, n_chunks)
    W1 = W[:, :C]
    W2 = W[:, C:]
    A = (W1 - W2).T
    B2 = W2.T
    return _tc_combine(features, sum_lo, sum_hi, counts,
                       A, B2[:H], B2[H:], b.reshape(1, C), V, C, H)


# speedup vs baseline: 21.1629x; 1.0094x over previous
import functools

import jax
import jax.numpy as jnp
from jax import lax
from jax.experimental import pallas as pl
from jax.experimental.pallas import tpu as pltpu
from jax.experimental.pallas import tpu_sc as plsc

NC = 2
NS = 16
CHUNK = 80


def _sc_segment_sum(feats2, edge_r, z_feat, z_cnt, ones8, V, H, n_chunks):
    r_stride, r_len = 624, 640
    assert (NS - 1) * r_stride + r_len == V
    mesh = plsc.VectorSubcoreMesh(core_axis_name="c", subcore_axis_name="s")

    @functools.partial(
        pl.kernel,
        mesh=mesh,
        compiler_params=pltpu.CompilerParams(use_tc_tiling_on_sc=False),
        out_type=(
            jax.ShapeDtypeStruct((V, H), jnp.float32),
            jax.ShapeDtypeStruct((V, H), jnp.float32),
            jax.ShapeDtypeStruct((V, 8), jnp.float32),
        ),
        scratch_types=[
            pltpu.VMEM((n_chunks, CHUNK), jnp.int32),
            pltpu.VMEM((n_chunks, CHUNK), jnp.int32),
            pltpu.VMEM((CHUNK, H), jnp.float32),
            pltpu.VMEM((CHUNK, H), jnp.float32),
            pltpu.VMEM((CHUNK, H), jnp.float32),
            pltpu.VMEM((CHUNK, H), jnp.float32),
            pltpu.VMEM((CHUNK, H), jnp.float32),
            pltpu.VMEM((CHUNK, 8), jnp.float32),
            pltpu.VMEM_SHARED((V, H), jnp.float32),
            pltpu.VMEM_SHARED((V, 8), jnp.float32),
        ] + [pltpu.SemaphoreType.DMA] * 10,
    )
    def sc_kernel(f2_hbm, edge_hbm, zf_hbm, zc_hbm, ones_hbm,
                  sum_lo_out, sum_hi_out, counts_out,
                  src_v, tgt_v,
                  rows0, rows1, rows2, rows3, rows4, ones_v,
                  acc_sh, cnt_sh, *sems):
        c = lax.axis_index("c")
        s = lax.axis_index("s")
        r0 = s * r_stride

        pltpu.sync_copy(zf_hbm, acc_sh.at[pl.ds(r0, r_len)])
        pltpu.sync_copy(edge_hbm.at[0, s], src_v)
        pltpu.sync_copy(edge_hbm.at[1, s], tgt_v)

        @pl.when(c == 0)
        def _():
            pltpu.sync_copy(zc_hbm, cnt_sh.at[pl.ds(r0, r_len)])
            pltpu.sync_copy(ones_hbm, ones_v)

        def idx_body(i, carry):
            for k in range(CHUNK // 16):
                sl = pl.ds(16 * k, 16)
                src_v[i, sl] = src_v[i, sl] * 2 + c
            return carry

        lax.fori_loop(0, n_chunks, idx_body, 0)

        plsc.subcore_barrier()

        D = 5
        bufs = (rows0, rows1, rows2, rows3, rows4)
        gsems, ssems = sems[0:D], sems[D:2 * D]
        csems = ssems

        def start_gather(i, buf, sem):
            pltpu.async_copy(f2_hbm.at[src_v.at[i]], buf, sem)

        def wait_gather(buf, sem):
            pltpu.make_async_copy(f2_hbm.at[pl.ds(0, CHUNK)], buf, sem).wait()

        def start_scatter(i, buf, sem):
            pltpu.async_copy(buf, acc_sh.at[tgt_v.at[i]], sem, add=True)

        def wait_scatter(buf, sem):
            pltpu.make_async_copy(buf, acc_sh.at[pl.ds(0, CHUNK)], sem).wait()

        def start_counts(i, sem):
            @pl.when(c == 0)
            def _():
                pltpu.async_copy(ones_v, cnt_sh.at[tgt_v.at[i]], sem,
                                 add=True)

        def wait_counts(sem):
            @pl.when(c == 0)
            def _():
                pltpu.make_async_copy(ones_v, cnt_sh.at[pl.ds(0, CHUNK)],
                                      sem).wait()

        def chunk_step(j, p, prefetch):
            wait_gather(bufs[p], gsems[p])
            start_scatter(j, bufs[p], ssems[p])
            start_counts(j, csems[p])
            if prefetch:
                p4 = (p + 4) % D
                wait_scatter(bufs[p4], ssems[p4])
                wait_counts(csems[p4])
                start_gather(j + 4, bufs[p4], gsems[p4])

        for j in range(D - 1):
            start_gather(j, bufs[j], gsems[j])
        wait_gather(bufs[0], gsems[0])
        start_scatter(0, bufs[0], ssems[0])
        start_counts(0, csems[0])
        start_gather(D - 1, bufs[D - 1], gsems[D - 1])

        def group_body(q, carry):
            j = D * q + 1
            for t in range(D):
                chunk_step(j + t, (1 + t) % D, True)
            return carry

        lax.fori_loop(0, (n_chunks - 1 - (D - 1)) // D, group_body, 0)
        for j in range(n_chunks - (D - 1), n_chunks):
            chunk_step(j, j % D, False)

        for p in range(D):
            wait_scatter(bufs[p], ssems[p])
            wait_counts(csems[p])

        plsc.subcore_barrier()

        @pl.when(c == 0)
        def _():
            pltpu.sync_copy(acc_sh.at[pl.ds(r0, r_len)],
                            sum_lo_out.at[pl.ds(r0, r_len)])
            pltpu.sync_copy(cnt_sh.at[pl.ds(r0, r_len)],
                            counts_out.at[pl.ds(r0, r_len)])

        @pl.when(c == 1)
        def _():
            pltpu.sync_copy(acc_sh.at[pl.ds(r0, r_len)],
                            sum_hi_out.at[pl.ds(r0, r_len)])

    return sc_kernel(feats2, edge_r, z_feat, z_cnt, ones8)


def _tc_combine(features, sum_lo, sum_hi, counts, A, B2L, B2H, bias, V, C, H):
    BLK = 1000
    grid = (V // BLK,)

    def body(f_ref, sl_ref, sh_ref, c_ref, a_ref, b2l_ref, b2h_ref,
             bias_ref, o_ref):
        cnt = c_ref[:, 0:1]
        mask = (cnt > 0.0).astype(jnp.float32)
        inv = 1.0 / jnp.maximum(cnt, 1.0)
        local = jnp.dot(f_ref[...], a_ref[...],
                        preferred_element_type=jnp.float32)
        nbr = (jnp.dot(sl_ref[...] * inv, b2l_ref[...],
                       preferred_element_type=jnp.float32)
               + jnp.dot(sh_ref[...] * inv, b2h_ref[...],
                         preferred_element_type=jnp.float32))
        o_ref[...] = mask * (local + bias_ref[...]) + nbr

    return pl.pallas_call(
        body,
        grid=grid,
        in_specs=[
            pl.BlockSpec((BLK, C), lambda i: (i, 0)),
            pl.BlockSpec((BLK, H), lambda i: (i, 0)),
            pl.BlockSpec((BLK, H), lambda i: (i, 0)),
            pl.BlockSpec((BLK, 8), lambda i: (i, 0)),
            pl.BlockSpec((C, C), lambda i: (0, 0)),
            pl.BlockSpec((H, C), lambda i: (0, 0)),
            pl.BlockSpec((H, C), lambda i: (0, 0)),
            pl.BlockSpec((1, C), lambda i: (0, 0)),
        ],
        out_specs=pl.BlockSpec((BLK, C), lambda i: (i, 0)),
        out_shape=jax.ShapeDtypeStruct((V, C), jnp.float32),
    )(features, sum_lo, sum_hi, counts, A, B2L, B2H, bias)


def kernel(features, edge_index, W, b):
    V, C = features.shape
    H = C // 2
    E = edge_index.shape[1]
    n_chunks = E // (NS * CHUNK)
    edge_r = edge_index.reshape(2, NS, n_chunks, CHUNK)
    feats2 = features.reshape(2 * V, H)
    z_feat = jnp.zeros((640, H), jnp.float32)
    z_cnt = jnp.zeros((640, 8), jnp.float32)
    ones8 = jnp.ones((CHUNK, 8), jnp.float32)
    sum_lo, sum_hi, counts = _sc_segment_sum(
        feats2, edge_r, z_feat, z_cnt, ones8, V, H, n_chunks)
    W1 = W[:, :C]
    W2 = W[:, C:]
    A = (W1 - W2).T
    B2 = W2.T
    return _tc_combine(features, sum_lo, sum_hi, counts,
                       A, B2[:H], B2[H:], b.reshape(1, C), V, C, H)
